# baseline (speedup 1.0000x reference)
import jax
import jax.numpy as jnp
from jax import lax
from jax.experimental import pallas as pl
from jax.experimental.pallas import tpu as pltpu
from jax.experimental.pallas import tpu_sc as plsc

L = 16
HW2 = 32
NBUF = 4
_HB = 16


def _tc_body(m1_ref, x0_ref, x1_ref, y1_ref):
    m1 = m1_ref[...] > 0
    y1_ref[...] = jnp.where(m1, x0_ref[...], x1_ref[...])


def _sc_body(x0_hbm, x1_hbm, bn2_hbm, thr_hbm, y2_hbm,
             w_v, thr_v, m2_v,
             a0, a1, a2, a3, b0, b1, b2, b3,
             sin0, sin1, sin2, sin3, sout0, sout1, sout2, sout3):
    nc = 2
    wid = lax.axis_index("s") * nc + lax.axis_index("c")
    B, H, W, C = x0_hbm.shape
    cpp = W // HW2
    n_chunks = (B * H * cpp) // (nc * L)
    c0 = wid * n_chunks

    abuf = (a0, a1, a2, a3)
    bbuf = (b0, b1, b2, b3)
    sin = (sin0, sin1, sin2, sin3)
    sout = (sout0, sout1, sout2, sout3)

    pltpu.sync_copy(thr_hbm, thr_v)
    thr = thr_v[...]
    pltpu.sync_copy(bn2_hbm, w_v)
    for k in range(C // L):
        m2_v[pl.ds(k * L, L)] = jnp.where(
            jnp.abs(w_v[pl.ds(k * L, L)]) >= thr, 1.0, 0.0)

    def _loc(c):
        p = c // cpp
        return p // H, p % H, (c % cpp) * HW2

    def _gather_start(c, s):
        b, h, w0 = _loc(c0 + c)
        pltpu.make_async_copy(
            x0_hbm.at[b, h, pl.ds(w0, HW2)], abuf[s], sin[s]).start()
        pltpu.make_async_copy(
            x1_hbm.at[b, h, pl.ds(w0, HW2)], bbuf[s], sin[s]).start()

    def _gather_wait(s):
        pltpu.make_async_copy(
            x0_hbm.at[0, 0, pl.ds(0, HW2)], abuf[s], sin[s]).wait()
        pltpu.make_async_copy(
            x1_hbm.at[0, 0, pl.ds(0, HW2)], bbuf[s], sin[s]).wait()

    def _scatter_start(c, s):
        b, h, w0 = _loc(c0 + c)
        pltpu.make_async_copy(
            bbuf[s], y2_hbm.at[b, h, pl.ds(w0, HW2)], sout[s]).start()

    def _scatter_wait(s):
        pltpu.make_async_copy(
            bbuf[s], y2_hbm.at[0, 0, pl.ds(0, HW2)], sout[s]).wait()

    def _compute(s):
        av_ref, bv_ref = abuf[s], bbuf[s]

        def kloop(k):
            sl = pl.ds(k * L, L)
            m2 = m2_v[sl] > 0.5

            def wloop(w):
                bv_ref[w, sl] = jnp.where(m2, bv_ref[w, sl], av_ref[w, sl])

            pl.loop(0, HW2)(wloop)

        pl.loop(0, C // L)(kloop)

    for s in range(NBUF - 1):
        _gather_start(s, s)

    def step(t):
        for s in range(NBUF):
            c = t * NBUF + s
            sprev = (s + NBUF - 1) % NBUF

            _gather_wait(s)
            _compute(s)
            _scatter_start(c, s)

            @pl.when(c + NBUF - 1 < n_chunks)
            def _():
                @pl.when(c >= 1)
                def _():
                    _scatter_wait(sprev)

                _gather_start(c + NBUF - 1, sprev)

    pl.loop(0, n_chunks // NBUF)(step)
    for s in range(NBUF):
        _scatter_wait(s)


def kernel(x0, x1, bn1_weight, bn2_weight, bn_threshold):
    B, C, H, W = x0.shape
    x0t = jnp.transpose(x0, (0, 2, 3, 1))
    x1t = jnp.transpose(x1, (0, 2, 3, 1))
    thr = jnp.full((L,), bn_threshold, dtype=jnp.float32)
    m1 = (jnp.abs(bn1_weight) >= bn_threshold).astype(jnp.float32)
    m1 = m1.reshape(1, 1, 1, C)

    mesh = plsc.VectorSubcoreMesh(core_axis_name="c", subcore_axis_name="s")
    chunk = pltpu.VMEM((HW2, C), jnp.float32)
    run_sc = pl.kernel(
        _sc_body,
        out_type=jax.ShapeDtypeStruct((B, H, W, C), jnp.float32),
        mesh=mesh,
        scratch_types=[
            pltpu.VMEM((C,), jnp.float32),
            pltpu.VMEM((L,), jnp.float32),
            pltpu.VMEM((C,), jnp.float32),
            chunk, chunk, chunk, chunk, chunk, chunk, chunk, chunk,
            pltpu.SemaphoreType.DMA, pltpu.SemaphoreType.DMA,
            pltpu.SemaphoreType.DMA, pltpu.SemaphoreType.DMA,
            pltpu.SemaphoreType.DMA, pltpu.SemaphoreType.DMA,
            pltpu.SemaphoreType.DMA, pltpu.SemaphoreType.DMA,
        ],
        compiler_params=pltpu.CompilerParams(use_tc_tiling_on_sc=True),
    )
    y2t = run_sc(x0t, x1t, bn2_weight, thr)

    HB = _HB
    y1t = pl.pallas_call(
        _tc_body,
        grid=(B, H // HB),
        in_specs=[
            pl.BlockSpec((1, 1, 1, C), lambda b, j: (0, 0, 0, 0)),
            pl.BlockSpec((1, HB, W, C), lambda b, j: (b, j, 0, 0)),
            pl.BlockSpec((1, HB, W, C), lambda b, j: (b, j, 0, 0)),
        ],
        out_specs=pl.BlockSpec((1, HB, W, C), lambda b, j: (b, j, 0, 0)),
        out_shape=jax.ShapeDtypeStruct((B, H, W, C), x0.dtype),
        compiler_params=pltpu.CompilerParams(
            dimension_semantics=("parallel", "parallel"),
        ),
    )(m1, x0t, x1t)

    return (jnp.transpose(y1t, (0, 3, 1, 2)), jnp.transpose(y2t, (0, 3, 1, 2)))


# speedup vs baseline: 1.0924x; 1.0924x over previous
import jax
import jax.numpy as jnp
from jax import lax
from jax.experimental import pallas as pl
from jax.experimental.pallas import tpu as pltpu
from jax.experimental.pallas import tpu_sc as plsc

L = 16
HW2 = 64
NBUF = 2


def _sc_body(x0_hbm, x1_hbm, bn1_hbm, bn2_hbm, thr_hbm,
             y1_hbm, y2_hbm,
             w_v, thr_v, m1_v, m2_v,
             a0, a1, b0, b1,
             sin0, sin1, sout0, sout1):
    nc = 2
    wid = lax.axis_index("s") * nc + lax.axis_index("c")
    B, H, W, C = x0_hbm.shape
    cpp = W // HW2
    n_chunks = (B * H * cpp) // (nc * L)
    c0 = wid * n_chunks

    abuf = (a0, a1)
    bbuf = (b0, b1)
    sin = (sin0, sin1)
    sout = (sout0, sout1)

    pltpu.sync_copy(thr_hbm, thr_v)
    thr = thr_v[...]
    pltpu.sync_copy(bn1_hbm, w_v)
    for k in range(C // L):
        m1_v[pl.ds(k * L, L)] = jnp.where(
            jnp.abs(w_v[pl.ds(k * L, L)]) >= thr, 1.0, 0.0)
    pltpu.sync_copy(bn2_hbm, w_v)
    for k in range(C // L):
        m2_v[pl.ds(k * L, L)] = jnp.where(
            jnp.abs(w_v[pl.ds(k * L, L)]) >= thr, 1.0, 0.0)

    def _loc(c):
        p = c // cpp
        return p // H, p % H, (c % cpp) * HW2

    def _gather_start(c, s):
        b, h, w0 = _loc(c0 + c)
        pltpu.make_async_copy(
            x0_hbm.at[b, h, pl.ds(w0, HW2)], abuf[s], sin[s]).start()
        pltpu.make_async_copy(
            x1_hbm.at[b, h, pl.ds(w0, HW2)], bbuf[s], sin[s]).start()

    def _gather_wait(s):
        pltpu.make_async_copy(
            x0_hbm.at[0, 0, pl.ds(0, HW2)], abuf[s], sin[s]).wait()
        pltpu.make_async_copy(
            x1_hbm.at[0, 0, pl.ds(0, HW2)], bbuf[s], sin[s]).wait()

    def _scatter_start(c, s):
        b, h, w0 = _loc(c0 + c)
        pltpu.make_async_copy(
            abuf[s], y1_hbm.at[b, h, pl.ds(w0, HW2)], sout[s]).start()
        pltpu.make_async_copy(
            bbuf[s], y2_hbm.at[b, h, pl.ds(w0, HW2)], sout[s]).start()

    def _scatter_wait(s):
        pltpu.make_async_copy(
            abuf[s], y1_hbm.at[0, 0, pl.ds(0, HW2)], sout[s]).wait()
        pltpu.make_async_copy(
            bbuf[s], y2_hbm.at[0, 0, pl.ds(0, HW2)], sout[s]).wait()

    def _compute(s):
        av_ref, bv_ref = abuf[s], bbuf[s]

        def kloop(k):
            sl = pl.ds(k * L, L)
            m1 = m1_v[sl] > 0.5
            m2 = m2_v[sl] > 0.5
            for w in range(HW2):
                av = av_ref[w, sl]
                bv = bv_ref[w, sl]
                av_ref[w, sl] = jnp.where(m1, av, bv)
                bv_ref[w, sl] = jnp.where(m2, bv, av)

        pl.loop(0, C // L)(kloop)

    for s in range(NBUF - 1):
        _gather_start(s, s)

    def step(t):
        for s in range(NBUF):
            c = t * NBUF + s
            sprev = (s + NBUF - 1) % NBUF

            _gather_wait(s)
            _compute(s)
            _scatter_start(c, s)

            @pl.when(c + NBUF - 1 < n_chunks)
            def _():
                @pl.when(c >= 1)
                def _():
                    _scatter_wait(sprev)

                _gather_start(c + NBUF - 1, sprev)

    pl.loop(0, n_chunks // NBUF)(step)
    for s in range(NBUF):
        _scatter_wait(s)


def kernel(x0, x1, bn1_weight, bn2_weight, bn_threshold):
    B, C, H, W = x0.shape
    x0t = jnp.transpose(x0, (0, 2, 3, 1))
    x1t = jnp.transpose(x1, (0, 2, 3, 1))
    thr = jnp.full((L,), bn_threshold, dtype=jnp.float32)

    mesh = plsc.VectorSubcoreMesh(core_axis_name="c", subcore_axis_name="s")
    chunk = pltpu.VMEM((HW2, C), jnp.float32)
    run = pl.kernel(
        _sc_body,
        out_type=[
            jax.ShapeDtypeStruct((B, H, W, C), jnp.float32),
            jax.ShapeDtypeStruct((B, H, W, C), jnp.float32),
        ],
        mesh=mesh,
        scratch_types=[
            pltpu.VMEM((C,), jnp.float32),
            pltpu.VMEM((L,), jnp.float32),
            pltpu.VMEM((C,), jnp.float32),
            pltpu.VMEM((C,), jnp.float32),
            chunk, chunk, chunk, chunk,
            pltpu.SemaphoreType.DMA, pltpu.SemaphoreType.DMA,
            pltpu.SemaphoreType.DMA, pltpu.SemaphoreType.DMA,
        ],
        compiler_params=pltpu.CompilerParams(use_tc_tiling_on_sc=True),
    )
    y1t, y2t = run(x0t, x1t, bn1_weight, bn2_weight, thr)
    return (jnp.transpose(y1t, (0, 3, 1, 2)), jnp.transpose(y2t, (0, 3, 1, 2)))


# speedup vs baseline: 1.4374x; 1.3159x over previous
import jax
import jax.numpy as jnp
from jax import lax
from jax.experimental import pallas as pl
from jax.experimental.pallas import tpu as pltpu
from jax.experimental.pallas import tpu_sc as plsc

L = 16
HW2 = 32
NBUF = 4


def _sc_body(x0_hbm, x1_hbm, bn1_hbm, bn2_hbm, thr_hbm,
             y1_hbm, y2_hbm,
             w_v, thr_v, m1_v, m2_v,
             a0, a1, a2, a3, b0, b1, b2, b3,
             sin0, sin1, sin2, sin3, sout0, sout1, sout2, sout3):
    nc = 2
    wid = lax.axis_index("s") * nc + lax.axis_index("c")
    B, H, W, C = x0_hbm.shape
    cpp = W // HW2
    n_chunks = (B * H * cpp) // (nc * L)
    c0 = wid * n_chunks

    abuf = (a0, a1, a2, a3)
    bbuf = (b0, b1, b2, b3)
    sin = (sin0, sin1, sin2, sin3)
    sout = (sout0, sout1, sout2, sout3)

    pltpu.sync_copy(thr_hbm, thr_v)
    thr = thr_v[...]
    pltpu.sync_copy(bn1_hbm, w_v)
    for k in range(C // L):
        m1_v[pl.ds(k * L, L)] = jnp.where(
            jnp.abs(w_v[pl.ds(k * L, L)]) >= thr, 1.0, 0.0)
    pltpu.sync_copy(bn2_hbm, w_v)
    for k in range(C // L):
        m2_v[pl.ds(k * L, L)] = jnp.where(
            jnp.abs(w_v[pl.ds(k * L, L)]) >= thr, 1.0, 0.0)

    def _loc(c):
        p = c // cpp
        return p // H, p % H, (c % cpp) * HW2

    def _gather_start(c, s):
        b, h, w0 = _loc(c0 + c)
        pltpu.make_async_copy(
            x0_hbm.at[b, h, pl.ds(w0, HW2)], abuf[s], sin[s]).start()
        pltpu.make_async_copy(
            x1_hbm.at[b, h, pl.ds(w0, HW2)], bbuf[s], sin[s]).start()

    def _gather_wait(s):
        pltpu.make_async_copy(
            x0_hbm.at[0, 0, pl.ds(0, HW2)], abuf[s], sin[s]).wait()
        pltpu.make_async_copy(
            x1_hbm.at[0, 0, pl.ds(0, HW2)], bbuf[s], sin[s]).wait()

    def _scatter_start(c, s):
        b, h, w0 = _loc(c0 + c)
        pltpu.make_async_copy(
            abuf[s], y1_hbm.at[b, h, pl.ds(w0, HW2)], sout[s]).start()
        pltpu.make_async_copy(
            bbuf[s], y2_hbm.at[b, h, pl.ds(w0, HW2)], sout[s]).start()

    def _scatter_wait(s):
        pltpu.make_async_copy(
            abuf[s], y1_hbm.at[0, 0, pl.ds(0, HW2)], sout[s]).wait()
        pltpu.make_async_copy(
            bbuf[s], y2_hbm.at[0, 0, pl.ds(0, HW2)], sout[s]).wait()

    def _compute(s):
        av_ref, bv_ref = abuf[s], bbuf[s]

        def kloop(k):
            sl = pl.ds(k * L, L)
            m1 = m1_v[sl] > 0.5
            m2 = m2_v[sl] > 0.5
            for w in range(HW2):
                av = av_ref[w, sl]
                bv = bv_ref[w, sl]
                av_ref[w, sl] = jnp.where(m1, av, bv)
                bv_ref[w, sl] = jnp.where(m2, bv, av)

        pl.loop(0, C // L)(kloop)

    for s in range(2):
        _gather_start(s, s)

    def step(t):
        for s in range(NBUF):
            c = t * NBUF + s
            snxt = (s + 2) % NBUF

            _gather_wait(s)
            _compute(s)
            _scatter_start(c, s)

            @pl.when(c + 2 < n_chunks)
            def _():
                @pl.when(c >= 2)
                def _():
                    _scatter_wait(snxt)

                _gather_start(c + 2, snxt)

    pl.loop(0, n_chunks // NBUF)(step)
    for s in range(NBUF):
        _scatter_wait(s)


def kernel(x0, x1, bn1_weight, bn2_weight, bn_threshold):
    B, C, H, W = x0.shape
    x0t = jnp.transpose(x0, (0, 2, 3, 1))
    x1t = jnp.transpose(x1, (0, 2, 3, 1))
    thr = jnp.full((L,), bn_threshold, dtype=jnp.float32)

    mesh = plsc.VectorSubcoreMesh(core_axis_name="c", subcore_axis_name="s")
    chunk = pltpu.VMEM((HW2, C), jnp.float32)
    run = pl.kernel(
        _sc_body,
        out_type=[
            jax.ShapeDtypeStruct((B, H, W, C), jnp.float32),
            jax.ShapeDtypeStruct((B, H, W, C), jnp.float32),
        ],
        mesh=mesh,
        scratch_types=[
            pltpu.VMEM((C,), jnp.float32),
            pltpu.VMEM((L,), jnp.float32),
            pltpu.VMEM((C,), jnp.float32),
            pltpu.VMEM((C,), jnp.float32),
            chunk, chunk, chunk, chunk, chunk, chunk, chunk, chunk,
            pltpu.SemaphoreType.DMA, pltpu.SemaphoreType.DMA,
            pltpu.SemaphoreType.DMA, pltpu.SemaphoreType.DMA,
            pltpu.SemaphoreType.DMA, pltpu.SemaphoreType.DMA,
            pltpu.SemaphoreType.DMA, pltpu.SemaphoreType.DMA,
        ],
        compiler_params=pltpu.CompilerParams(use_tc_tiling_on_sc=True),
    )
    y1t, y2t = run(x0t, x1t, bn1_weight, bn2_weight, thr)
    return (jnp.transpose(y1t, (0, 3, 1, 2)), jnp.transpose(y2t, (0, 3, 1, 2)))
